# Initial kernel scaffold; baseline (speedup 1.0000x reference)
import jax
import jax.numpy as jnp
from jax.experimental import pallas as pl


def kernel(x, edge_index, W1a, b1a, W1b, b1b, W2a, b2a, W2b, b2b):
    raise NotImplementedError("write your pallas kernel here")


# speedup vs baseline: 2.7066x; 2.7066x over previous
import functools

import jax
import jax.numpy as jnp
from jax import lax
from jax.experimental import pallas as pl
from jax.experimental.pallas import tpu as pltpu
from jax.experimental.pallas import tpu_sc as plsc

N = 10000
E = 320000
D = 128

NC = 2
NS = 16
NW = NC * NS
CH = 128
NCHUNK = 80
EPAD = NW * NCHUNK * CH
NPAD = 10240
RPT = NPAD // NS


def _agg_body(h_hbm, src_hbm, dst_hbm, out_hbm, sidx, didx, rows, accum, sem):
    c = lax.axis_index("c")
    s = lax.axis_index("s")
    w = c * NS + s

    zeros16 = jnp.zeros((16,), jnp.float32)

    def zfill(i, carry):
        r = i // (D // 16)
        col = (i % (D // 16)) * 16
        rows[r, pl.ds(col, 16)] = zeros16
        return carry

    lax.fori_loop(0, CH * (D // 16), zfill, 0)
    for k in range(RPT // CH):
        pltpu.sync_copy(rows, accum.at[pl.ds(s * RPT + k * CH, CH)])
    plsc.subcore_barrier()

    pltpu.sync_copy(src_hbm.at[w], sidx)
    pltpu.sync_copy(dst_hbm.at[w], didx)

    def body(j, carry):
        pltpu.async_copy(h_hbm.at[sidx.at[j]], rows, sem).wait()
        pltpu.sync_copy(rows, accum.at[didx.at[j]], add=True)
        return carry

    lax.fori_loop(0, NCHUNK, body, 0)
    plsc.subcore_barrier()

    pltpu.sync_copy(accum.at[pl.ds(s * RPT, RPT)],
                    out_hbm.at[c, pl.ds(s * RPT, RPT)])


_agg_call = pl.kernel(
    _agg_body,
    out_type=jax.ShapeDtypeStruct((NC, NPAD, D), jnp.float32),
    mesh=plsc.VectorSubcoreMesh(core_axis_name="c", subcore_axis_name="s"),
    scratch_types=[
        pltpu.VMEM((NCHUNK, CH), jnp.int32),
        pltpu.VMEM((NCHUNK, CH), jnp.int32),
        pltpu.VMEM((CH, D), jnp.float32),
        pltpu.VMEM_SHARED((NPAD, D), jnp.float32),
        pltpu.SemaphoreType.DMA,
    ],
)


BLK = 1000


def _mlp_body(x_ref, a0_ref, a1_ref, wa_ref, ba_ref, wb_ref, bb_ref, o_ref,
              *, final_relu):
    z = x_ref[...] + a0_ref[...] + a1_ref[...]
    z = jnp.dot(z, wa_ref[...], preferred_element_type=jnp.float32)
    z = jnp.maximum(z + ba_ref[...], 0.0)
    z = jnp.dot(z, wb_ref[...], preferred_element_type=jnp.float32)
    z = z + bb_ref[...]
    if final_relu:
        z = jnp.maximum(z, 0.0)
    o_ref[...] = z


def _mlp(x, a0, a1, Wa, ba, Wb, bb, final_relu):
    row_spec = pl.BlockSpec((BLK, D), lambda i: (i, 0))
    full_spec = pl.BlockSpec((D, D), lambda i: (0, 0))
    bias_spec = pl.BlockSpec((1, D), lambda i: (0, 0))
    return pl.pallas_call(
        functools.partial(_mlp_body, final_relu=final_relu),
        grid=(N // BLK,),
        in_specs=[row_spec, row_spec, row_spec, full_spec, bias_spec,
                  full_spec, bias_spec],
        out_specs=row_spec,
        out_shape=jax.ShapeDtypeStruct((N, D), jnp.float32),
    )(x, a0, a1, Wa, ba.reshape(1, D), Wb, bb.reshape(1, D))


def _pad_nodes(h):
    return jnp.pad(h, ((0, NPAD - N), (0, 0)))


def kernel(x, edge_index, W1a, b1a, W1b, b1b, W2a, b2a, W2b, b2b):
    npad = EPAD - E
    src = jnp.concatenate(
        [edge_index[0], jnp.full((npad,), N, jnp.int32)]).reshape(
            NW, NCHUNK, CH)
    dst = jnp.concatenate(
        [edge_index[1], jnp.zeros((npad,), jnp.int32)]).reshape(
            NW, NCHUNK, CH)

    agg1 = _agg_call(_pad_nodes(x), src, dst)
    h = _mlp(x, agg1[0, :N], agg1[1, :N], W1a, b1a, W1b, b1b,
             final_relu=True)
    agg2 = _agg_call(_pad_nodes(h), src, dst)
    out = _mlp(h, agg2[0, :N], agg2[1, :N], W2a, b2a, W2b, b2b,
               final_relu=False)
    return out


# speedup vs baseline: 3.0724x; 1.1351x over previous
import functools

import jax
import jax.numpy as jnp
from jax import lax
from jax.experimental import pallas as pl
from jax.experimental.pallas import tpu as pltpu
from jax.experimental.pallas import tpu_sc as plsc

N = 10000
E = 320000
D = 128

NC = 2
NS = 16
NW = NC * NS
CH = 128
NCHUNK = 80
EPAD = NW * NCHUNK * CH
NPAD = 10112
RPT = NPAD // NS


def _agg_body(h_hbm, src_hbm, dst_hbm, out_hbm, sidx, dring0, dring1,
              rows0, rows1, accum, sem0, sem1, semd0, semd1, semi):
    c = lax.axis_index("c")
    s = lax.axis_index("s")
    w = c * NS + s

    pltpu.async_copy(src_hbm.at[w], sidx, semi)

    zeros16 = jnp.zeros((16,), jnp.float32)

    def zfill(i, carry):
        r = i // (D // 16)
        col = (i % (D // 16)) * 16
        rows0[r, pl.ds(col, 16)] = zeros16
        return carry

    lax.fori_loop(0, CH * (D // 16), zfill, 0)
    for k in range(RPT // CH):
        pltpu.sync_copy(rows0, accum.at[pl.ds(s * RPT + k * CH, CH)])
    rem = RPT - (RPT // CH) * CH
    if rem:
        pltpu.sync_copy(rows0.at[pl.ds(0, rem)],
                        accum.at[pl.ds(s * RPT + (RPT // CH) * CH, rem)])

    pltpu.async_copy(dst_hbm.at[w, pl.ds(0, 1)], dring0, semd0)
    pltpu.async_copy(dst_hbm.at[w, pl.ds(1, 1)], dring1, semd1)
    pltpu.make_async_copy(src_hbm.at[w], sidx, semi).wait()
    pltpu.async_copy(h_hbm.at[sidx.at[0]], rows0, sem0)
    pltpu.async_copy(h_hbm.at[sidx.at[1]], rows1, sem1)
    plsc.subcore_barrier()

    def body(i, carry):
        j = 2 * i
        pltpu.make_async_copy(h_hbm.at[sidx.at[j]], rows0, sem0).wait()
        pltpu.make_async_copy(dst_hbm.at[w, pl.ds(j, 1)], dring0,
                              semd0).wait()
        pltpu.sync_copy(rows0, accum.at[dring0.at[0]], add=True)
        pltpu.async_copy(dst_hbm.at[w, pl.ds(j + 2, 1)], dring0, semd0)
        pltpu.async_copy(h_hbm.at[sidx.at[j + 2]], rows0, sem0)
        pltpu.make_async_copy(h_hbm.at[sidx.at[j + 1]], rows1, sem1).wait()
        pltpu.make_async_copy(dst_hbm.at[w, pl.ds(j + 1, 1)], dring1,
                              semd1).wait()
        pltpu.sync_copy(rows1, accum.at[dring1.at[0]], add=True)
        pltpu.async_copy(dst_hbm.at[w, pl.ds(j + 3, 1)], dring1, semd1)
        pltpu.async_copy(h_hbm.at[sidx.at[j + 3]], rows1, sem1)
        return carry

    lax.fori_loop(0, NCHUNK // 2 - 1, body, 0)
    j = NCHUNK - 2
    pltpu.make_async_copy(h_hbm.at[sidx.at[j]], rows0, sem0).wait()
    pltpu.make_async_copy(dst_hbm.at[w, pl.ds(j, 1)], dring0, semd0).wait()
    pltpu.sync_copy(rows0, accum.at[dring0.at[0]], add=True)
    pltpu.make_async_copy(h_hbm.at[sidx.at[j + 1]], rows1, sem1).wait()
    pltpu.make_async_copy(dst_hbm.at[w, pl.ds(j + 1, 1)], dring1,
                          semd1).wait()
    pltpu.sync_copy(rows1, accum.at[dring1.at[0]], add=True)
    plsc.subcore_barrier()

    pltpu.sync_copy(accum.at[pl.ds(s * RPT, RPT)],
                    out_hbm.at[c, pl.ds(s * RPT, RPT)])


_agg_call = pl.kernel(
    _agg_body,
    out_type=jax.ShapeDtypeStruct((NC, NPAD, D), jnp.float32),
    mesh=plsc.VectorSubcoreMesh(core_axis_name="c", subcore_axis_name="s"),
    scratch_types=[
        pltpu.VMEM((NCHUNK, CH), jnp.int32),
        pltpu.VMEM((1, CH), jnp.int32),
        pltpu.VMEM((1, CH), jnp.int32),
        pltpu.VMEM((CH, D), jnp.float32),
        pltpu.VMEM((CH, D), jnp.float32),
        pltpu.VMEM_SHARED((NPAD, D), jnp.float32),
        pltpu.SemaphoreType.DMA,
        pltpu.SemaphoreType.DMA,
        pltpu.SemaphoreType.DMA,
        pltpu.SemaphoreType.DMA,
        pltpu.SemaphoreType.DMA,
    ],
)


BLK = 1000


def _mlp_body(x_ref, a0_ref, a1_ref, wa_ref, ba_ref, wb_ref, bb_ref, o_ref,
              *, final_relu):
    z = x_ref[...] + a0_ref[...] + a1_ref[...]
    z = jnp.dot(z, wa_ref[...], preferred_element_type=jnp.float32)
    z = jnp.maximum(z + ba_ref[...], 0.0)
    z = jnp.dot(z, wb_ref[...], preferred_element_type=jnp.float32)
    z = z + bb_ref[...]
    if final_relu:
        z = jnp.maximum(z, 0.0)
    o_ref[...] = z


def _mlp(x, a0, a1, Wa, ba, Wb, bb, final_relu):
    row_spec = pl.BlockSpec((BLK, D), lambda i: (i, 0))
    full_spec = pl.BlockSpec((D, D), lambda i: (0, 0))
    bias_spec = pl.BlockSpec((1, D), lambda i: (0, 0))
    return pl.pallas_call(
        functools.partial(_mlp_body, final_relu=final_relu),
        grid=(N // BLK,),
        in_specs=[row_spec, row_spec, row_spec, full_spec, bias_spec,
                  full_spec, bias_spec],
        out_specs=row_spec,
        out_shape=jax.ShapeDtypeStruct((N, D), jnp.float32),
    )(x, a0, a1, Wa, ba.reshape(1, D), Wb, bb.reshape(1, D))


def _pad_nodes(h):
    return jnp.pad(h, ((0, NPAD - N), (0, 0)))


def kernel(x, edge_index, W1a, b1a, W1b, b1b, W2a, b2a, W2b, b2b):
    npad = EPAD - E
    src = jnp.concatenate(
        [edge_index[0], jnp.full((npad,), N, jnp.int32)]).reshape(
            NW, NCHUNK, CH)
    dst = jnp.concatenate(
        [edge_index[1], jnp.zeros((npad,), jnp.int32)]).reshape(
            NW, NCHUNK, CH)

    agg1 = _agg_call(_pad_nodes(x), src, dst)
    h = _mlp(x, agg1[0, :N], agg1[1, :N], W1a, b1a, W1b, b1b,
             final_relu=True)
    agg2 = _agg_call(_pad_nodes(h), src, dst)
    out = _mlp(h, agg2[0, :N], agg2[1, :N], W2a, b2a, W2b, b2b,
               final_relu=False)
    return out
